# baseline (speedup 1.0000x reference)
import jax
import jax.numpy as jnp
from jax.experimental import pallas as pl
from jax.experimental.pallas import tpu as pltpu

_BM = 1024
_MT = 4096 // _BM
_NSTAGE = 24


def _fused_kernel(x_ref, w1_ref, w2_ref, w3_ref, b1_ref, b2_ref, b3_ref,
                  o_ref, h1_ref, h2_ref):
    s = pl.program_id(1)

    @pl.when(s < 4)
    def _l1():
        xb = x_ref[...].astype(jnp.bfloat16)
        wb = w1_ref[...].astype(jnp.bfloat16)
        acc = jnp.dot(xb, wb, preferred_element_type=jnp.float32)
        acc = jnp.maximum(acc + b1_ref[0], 0.0)
        h1_ref[s] = acc.astype(jnp.bfloat16)

    @pl.when((s >= 4) & (s < 20))
    def _l2():
        acc = None
        for j in range(4):
            wb = w2_ref[j].astype(jnp.bfloat16)
            d = jnp.dot(h1_ref[j], wb, preferred_element_type=jnp.float32)
            acc = d if acc is None else acc + d
        acc = jnp.maximum(acc + b2_ref[0], 0.0)
        h2_ref[s - 4] = acc.astype(jnp.bfloat16)

    @pl.when(s >= 20)
    def _l3():
        k = s - 20
        acc = None
        for j in range(4):
            wb = w3_ref[j].astype(jnp.bfloat16)
            d = jnp.dot(h2_ref[4 * k + j], wb, preferred_element_type=jnp.float32)
            acc = d if acc is None else acc + d

        @pl.when(s == 20)
        def _init():
            o_ref[...] = acc + b3_ref[...]

        @pl.when(s > 20)
        def _accum():
            o_ref[...] += acc


def kernel(x, W1, b1, W2, b2, W3, b3, interpret=False):
    w2r = W2.reshape(4, 1024, 4096)
    w3r = W3.reshape(16, 256, 1000)
    b1r = b1.reshape(4, 1, 1024)
    b2r = b2.reshape(16, 1, 256)
    b3r = b3.reshape(1, 1000)
    return pl.pallas_call(
        _fused_kernel,
        grid=(_MT, _NSTAGE),
        in_specs=[
            pl.BlockSpec((_BM, 1024), lambda m, s: (m, 0)),
            pl.BlockSpec((1024, 1024),
                         lambda m, s: (0, jnp.minimum(s, 3))),
            pl.BlockSpec((4, 1024, 256),
                         lambda m, s: (0, 0, jnp.clip(s - 4, 0, 15))),
            pl.BlockSpec((4, 256, 1000),
                         lambda m, s: (jnp.clip(s - 20, 0, 3), 0, 0)),
            pl.BlockSpec((1, 1, 1024), lambda m, s: (jnp.minimum(s, 3), 0, 0)),
            pl.BlockSpec((1, 1, 256), lambda m, s: (jnp.clip(s - 4, 0, 15), 0, 0)),
            pl.BlockSpec((1, 1000), lambda m, s: (0, 0)),
        ],
        out_specs=pl.BlockSpec((_BM, 1000), lambda m, s: (m, 0)),
        out_shape=jax.ShapeDtypeStruct((4096, 1000), jnp.float32),
        scratch_shapes=[
            pltpu.VMEM((4, _BM, 1024), jnp.bfloat16),
            pltpu.VMEM((16, _BM, 256), jnp.bfloat16),
        ],
        compiler_params=pltpu.CompilerParams(
            dimension_semantics=("arbitrary", "arbitrary"),
            vmem_limit_bytes=63 * 1024 * 1024,
        ),
        interpret=interpret,
    )(x, W1, w2r, w3r, b1r, b2r, b3r)


# speedup vs baseline: 1.0548x; 1.0548x over previous
import jax
import jax.numpy as jnp
from jax.experimental import pallas as pl
from jax.experimental.pallas import tpu as pltpu

_BM = 1024
_MT = 4096 // _BM
_NSTAGE = 24


def _ds(i, width):
    return pl.ds(pl.multiple_of(i * width, width), width)


def _fused_kernel(x_ref, w1_ref, w2_ref, w3_ref, b1_ref, b2_ref, b3_ref,
                  o_ref, h1_ref, h2_ref):
    s = pl.program_id(1)

    @pl.when(s < 4)
    def _l1():
        xb = x_ref[...].astype(jnp.bfloat16)
        wb = w1_ref[...].astype(jnp.bfloat16)
        acc = jnp.dot(xb, wb, preferred_element_type=jnp.float32)
        acc = jnp.maximum(acc + b1_ref[0], 0.0)
        h1_ref[:, _ds(s, 1024)] = acc.astype(jnp.bfloat16)

    @pl.when((s >= 4) & (s < 20))
    def _l2():
        wb = w2_ref[...].astype(jnp.bfloat16)
        acc = jnp.dot(h1_ref[...], wb, preferred_element_type=jnp.float32)
        acc = jnp.maximum(acc + b2_ref[0], 0.0)
        h2_ref[:, _ds(s - 4, 256)] = acc.astype(jnp.bfloat16)

    @pl.when(s >= 20)
    def _l3():
        k = s - 20
        wb = w3_ref[...].astype(jnp.bfloat16)
        acc = jnp.dot(h2_ref[:, _ds(k, 1024)], wb,
                      preferred_element_type=jnp.float32)

        @pl.when(s == 20)
        def _init():
            o_ref[...] = acc + b3_ref[...]

        @pl.when(s > 20)
        def _accum():
            o_ref[...] += acc


def kernel(x, W1, b1, W2, b2, W3, b3, interpret=False):
    b1r = b1.reshape(4, 1, 1024)
    b2r = b2.reshape(16, 1, 256)
    b3r = b3.reshape(1, 1000)
    return pl.pallas_call(
        _fused_kernel,
        grid=(_MT, _NSTAGE),
        in_specs=[
            pl.BlockSpec((_BM, 1024), lambda m, s: (m, 0)),
            pl.BlockSpec((1024, 1024),
                         lambda m, s: (0, jnp.minimum(s, 3))),
            pl.BlockSpec((4096, 256),
                         lambda m, s: (0, jnp.clip(s - 4, 0, 15))),
            pl.BlockSpec((1024, 1000),
                         lambda m, s: (jnp.clip(s - 20, 0, 3), 0)),
            pl.BlockSpec((1, 1, 1024), lambda m, s: (jnp.minimum(s, 3), 0, 0)),
            pl.BlockSpec((1, 1, 256), lambda m, s: (jnp.clip(s - 4, 0, 15), 0, 0)),
            pl.BlockSpec((1, 1000), lambda m, s: (0, 0)),
        ],
        out_specs=pl.BlockSpec((_BM, 1000), lambda m, s: (m, 0)),
        out_shape=jax.ShapeDtypeStruct((4096, 1000), jnp.float32),
        scratch_shapes=[
            pltpu.VMEM((_BM, 4096), jnp.bfloat16),
            pltpu.VMEM((_BM, 4096), jnp.bfloat16),
        ],
        compiler_params=pltpu.CompilerParams(
            dimension_semantics=("arbitrary", "arbitrary"),
            vmem_limit_bytes=63 * 1024 * 1024,
        ),
        interpret=interpret,
    )(x, W1, W2, W3, b1r, b2r, b3r)
